# Initial kernel scaffold; baseline (speedup 1.0000x reference)
import jax
import jax.numpy as jnp
from jax.experimental import pallas as pl


def kernel(features, edge_index, s):
    raise NotImplementedError("write your pallas kernel here")


# speedup vs baseline: 9.7537x; 9.7537x over previous
import functools

import jax
import jax.numpy as jnp
from jax import lax
from jax.experimental import pallas as pl
from jax.experimental.pallas import tpu as pltpu
from jax.experimental.pallas import tpu_sc as plsc

N = 10000
E = 320000
D = 128
K = 10

NC = 2
NS = 16
NW = NC * NS
DH = D // NC

C = 125
ES = E // NS
NCH = ES // C

EW = E // NW
NCHD = EW // C

RT = N // NS
NZ = RT // C
DT = 624
DREM = N - NS * DT

_mesh = plsc.VectorSubcoreMesh(core_axis_name="c", subcore_axis_name="s")


@functools.partial(
    pl.kernel,
    out_type=jax.ShapeDtypeStruct((NC, N, 16), jnp.float32),
    mesh=_mesh,
    scratch_types=[
        pltpu.VMEM((NCHD, C), jnp.int32),
        pltpu.VMEM((C, 16), jnp.float32),
        pltpu.VMEM_SHARED((N, 16), jnp.float32),
    ],
    compiler_params=pltpu.CompilerParams(use_tc_tiling_on_sc=False),
)
def _deg_kernel(dst_hbm, out_hbm, dstv, onesv, acc):
    c = lax.axis_index("c")
    s = lax.axis_index("s")
    w = s * NC + c
    pltpu.sync_copy(dst_hbm.at[w], dstv)

    def _zrow(r, _):
        onesv[r, :] = jnp.zeros((16,), jnp.float32)
        return 0
    lax.fori_loop(0, C, _zrow, 0)
    for q in range(NZ):
        pltpu.sync_copy(onesv, acc.at[pl.ds(s * RT + q * C, C)])
    plsc.subcore_barrier()

    def _orow(r, _):
        onesv[r, :] = jnp.ones((16,), jnp.float32)
        return 0
    lax.fori_loop(0, C, _orow, 0)

    def _chunk(j, _):
        pltpu.sync_copy(onesv, acc.at[dstv.at[j]], add=True)
        return 0
    lax.fori_loop(0, NCHD, _chunk, 0)
    plsc.subcore_barrier()

    pltpu.sync_copy(acc.at[pl.ds(s * DT, DT)],
                    out_hbm.at[c, pl.ds(s * DT, DT)])

    @pl.when(s == NS - 1)
    def _():
        pltpu.sync_copy(acc.at[pl.ds(NS * DT, DREM)],
                        out_hbm.at[c, pl.ds(NS * DT, DREM)])


@functools.partial(
    pl.kernel,
    out_type=jax.ShapeDtypeStruct((NC, N, DH), jnp.float32),
    mesh=_mesh,
    scratch_types=[
        pltpu.VMEM((NCH, C), jnp.int32),
        pltpu.VMEM((NCH, C), jnp.int32),
        pltpu.VMEM((C, DH), jnp.float32),
        pltpu.VMEM((C, DH), jnp.float32),
        pltpu.VMEM_SHARED((N, DH), jnp.float32),
        pltpu.SemaphoreType.DMA,
        pltpu.SemaphoreType.DMA,
    ],
    compiler_params=pltpu.CompilerParams(use_tc_tiling_on_sc=False),
)
def _hop_kernel(g_hbm, src_hbm, dst_hbm, out_hbm,
                srcv, dstv, buf0, buf1, acc, sem0, sem1):
    c = lax.axis_index("c")
    s = lax.axis_index("s")
    pltpu.sync_copy(src_hbm.at[s], srcv)
    pltpu.sync_copy(dst_hbm.at[s], dstv)
    gc = g_hbm.at[c]

    def _zrow(r, _):
        for d8 in range(DH // 16):
            buf0[r, pl.ds(d8 * 16, 16)] = jnp.zeros((16,), jnp.float32)
        return 0
    lax.fori_loop(0, C, _zrow, 0)
    for q in range(NZ):
        pltpu.sync_copy(buf0, acc.at[pl.ds(s * RT + q * C, C)])
    plsc.subcore_barrier()

    def _gather(j, buf, sem):
        return pltpu.make_async_copy(gc.at[srcv.at[j]], buf, sem)

    _gather(0, buf0, sem0).start()

    def _pair(i, _):
        j0 = 2 * i
        _gather(j0 + 1, buf1, sem1).start()
        _gather(j0, buf0, sem0).wait()
        pltpu.sync_copy(buf0, acc.at[dstv.at[j0]], add=True)

        @pl.when(i < NCH // 2 - 1)
        def _():
            _gather(j0 + 2, buf0, sem0).start()

        _gather(j0 + 1, buf1, sem1).wait()
        pltpu.sync_copy(buf1, acc.at[dstv.at[j0 + 1]], add=True)
        return 0

    lax.fori_loop(0, NCH // 2, _pair, 0)
    plsc.subcore_barrier()

    pltpu.sync_copy(acc.at[pl.ds(s * DT, DT)],
                    out_hbm.at[c, pl.ds(s * DT, DT)])

    @pl.when(s == NS - 1)
    def _():
        pltpu.sync_copy(acc.at[pl.ds(NS * DT, DREM)],
                        out_hbm.at[c, pl.ds(NS * DT, DREM)])


_NB = 2000
_GRID = N // _NB


def _tc_init_body(feat_ref, degp_ref, s_ref,
                  g_ref, out_ref, invd_ref, sqd_ref):
    degs = jnp.maximum(degp_ref[0, :, 0:1] + degp_ref[1, :, 0:1], 1.0)
    invd_ref[...] = 1.0 / degs
    sqd_ref[...] = jnp.sqrt(degs)
    norm = lax.rsqrt(degs)
    h = feat_ref[...]
    g = h * norm
    g_ref[0] = g[:, :DH]
    g_ref[1] = g[:, DH:]
    t = jnp.dot(h, s_ref[...], preferred_element_type=jnp.float32)
    out_ref[...] = jax.nn.sigmoid(t) * h


def _tc_combine_body(part_ref, invd_ref, sqd_ref, s_ref, prev_ref,
                     g_ref, out_ref):
    seg = jnp.concatenate([part_ref[0], part_ref[1]], axis=-1)
    g = seg * invd_ref[...]
    h = g * sqd_ref[...]
    t = jnp.dot(h, s_ref[...], preferred_element_type=jnp.float32)
    g_ref[0] = g[:, :DH]
    g_ref[1] = g[:, DH:]
    out_ref[...] = prev_ref[...] + jax.nn.sigmoid(t) * h


_tc_init = pl.pallas_call(
    _tc_init_body,
    grid=(_GRID,),
    in_specs=[
        pl.BlockSpec((_NB, D), lambda i: (i, 0)),
        pl.BlockSpec((NC, _NB, 16), lambda i: (0, i, 0)),
        pl.BlockSpec((D, 1), lambda i: (0, 0)),
    ],
    out_specs=[
        pl.BlockSpec((NC, _NB, DH), lambda i: (0, i, 0)),
        pl.BlockSpec((_NB, D), lambda i: (i, 0)),
        pl.BlockSpec((_NB, 1), lambda i: (i, 0)),
        pl.BlockSpec((_NB, 1), lambda i: (i, 0)),
    ],
    out_shape=[
        jax.ShapeDtypeStruct((NC, N, DH), jnp.float32),
        jax.ShapeDtypeStruct((N, D), jnp.float32),
        jax.ShapeDtypeStruct((N, 1), jnp.float32),
        jax.ShapeDtypeStruct((N, 1), jnp.float32),
    ],
)

_tc_combine = pl.pallas_call(
    _tc_combine_body,
    grid=(_GRID,),
    in_specs=[
        pl.BlockSpec((NC, _NB, DH), lambda i: (0, i, 0)),
        pl.BlockSpec((_NB, 1), lambda i: (i, 0)),
        pl.BlockSpec((_NB, 1), lambda i: (i, 0)),
        pl.BlockSpec((D, 1), lambda i: (0, 0)),
        pl.BlockSpec((_NB, D), lambda i: (i, 0)),
    ],
    out_specs=[
        pl.BlockSpec((NC, _NB, DH), lambda i: (0, i, 0)),
        pl.BlockSpec((_NB, D), lambda i: (i, 0)),
    ],
    out_shape=[
        jax.ShapeDtypeStruct((NC, N, DH), jnp.float32),
        jax.ShapeDtypeStruct((N, D), jnp.float32),
    ],
)


def kernel(features, edge_index, s):
    src = edge_index[0].reshape(NS, NCH, C)
    dst = edge_index[1].reshape(NS, NCH, C)
    srcd = edge_index[0].reshape(NW, NCHD, C)
    dstd = edge_index[1].reshape(NW, NCHD, C)
    del srcd
    degparts = _deg_kernel(dstd)
    g, out, invd, sqd = _tc_init(features, degparts, s)
    for _ in range(K):
        part = _hop_kernel(g, src, dst)
        g, out = _tc_combine(part, invd, sqd, s, out)
    return out


# speedup vs baseline: 11.9802x; 1.2283x over previous
import functools

import jax
import jax.numpy as jnp
from jax import lax
from jax.experimental import pallas as pl
from jax.experimental.pallas import tpu as pltpu
from jax.experimental.pallas import tpu_sc as plsc

N = 10000
E = 320000
D = 128
K = 10

NC = 2
NS = 16
NW = NC * NS
DH = D // NC

C = 125
ES = E // NS
NCH = ES // C

EW = E // NW
NCHD = EW // C

RT = N // NS
NZ = RT // C
DT = 624
DREM = N - NS * DT
RPAD = 640

_mesh = plsc.VectorSubcoreMesh(core_axis_name="c", subcore_axis_name="s")
_sc_params = pltpu.CompilerParams(use_tc_tiling_on_sc=False)


@functools.partial(
    pl.kernel,
    out_type=jax.ShapeDtypeStruct((NC, N, 16), jnp.float32),
    mesh=_mesh,
    scratch_types=[
        pltpu.VMEM((NCHD, C), jnp.int32),
        pltpu.VMEM((C, 16), jnp.float32),
        pltpu.VMEM_SHARED((N, 16), jnp.float32),
    ],
    compiler_params=_sc_params,
)
def _deg_kernel(dst_hbm, out_hbm, dstv, onesv, acc):
    c = lax.axis_index("c")
    s = lax.axis_index("s")
    w = s * NC + c
    pltpu.sync_copy(dst_hbm.at[w], dstv)

    def _zrow(r, _):
        onesv[r, :] = jnp.zeros((16,), jnp.float32)
        return 0
    lax.fori_loop(0, C, _zrow, 0)
    for q in range(NZ):
        pltpu.sync_copy(onesv, acc.at[pl.ds(s * RT + q * C, C)])
    plsc.subcore_barrier()

    def _orow(r, _):
        onesv[r, :] = jnp.ones((16,), jnp.float32)
        return 0
    lax.fori_loop(0, C, _orow, 0)

    def _chunk(j, _):
        pltpu.sync_copy(onesv, acc.at[dstv.at[j]], add=True)
        return 0
    lax.fori_loop(0, NCHD, _chunk, 0)
    plsc.subcore_barrier()

    pltpu.sync_copy(acc.at[pl.ds(s * DT, DT)],
                    out_hbm.at[c, pl.ds(s * DT, DT)])

    @pl.when(s == NS - 1)
    def _():
        pltpu.sync_copy(acc.at[pl.ds(NS * DT, DREM)],
                        out_hbm.at[c, pl.ds(NS * DT, DREM)])


@functools.partial(
    pl.kernel,
    out_type=jax.ShapeDtypeStruct((K, NC, N, DH), jnp.float32),
    mesh=_mesh,
    scratch_types=[
        pltpu.VMEM((NCH, C), jnp.int32),
        pltpu.VMEM((NCH, C), jnp.int32),
        pltpu.VMEM((C, DH), jnp.float32),
        pltpu.VMEM((C, DH), jnp.float32),
        pltpu.VMEM((C, DH), jnp.float32),
        pltpu.VMEM((C, DH), jnp.float32),
        pltpu.VMEM((C, DH), jnp.float32),
        pltpu.VMEM((RPAD,), jnp.float32),
        pltpu.SMEM((RPAD,), jnp.float32),
        pltpu.VMEM_SHARED((N, DH), jnp.float32),
        pltpu.SemaphoreType.DMA,
        pltpu.SemaphoreType.DMA,
        pltpu.SemaphoreType.DMA,
        pltpu.SemaphoreType.DMA,
        pltpu.SemaphoreType.DMA,
        pltpu.SemaphoreType.DMA,
        pltpu.SemaphoreType.DMA,
        pltpu.SemaphoreType.DMA,
    ],
    compiler_params=_sc_params,
)
def _multihop_kernel(g0_hbm, src_hbm, dst_hbm, invd_hbm, gall_hbm,
                     srcv, dstv, b0, b1, b2, b3, wbuf, invdv, invds, acc,
                     sg0, sg1, sg2, sg3, ss0, ss1, ss2, ss3):
    c = lax.axis_index("c")
    s = lax.axis_index("s")
    bufs = (b0, b1, b2, b3)
    gsems = (sg0, sg1, sg2, sg3)
    ssems = (ss0, ss1, ss2, ss3)

    pltpu.sync_copy(src_hbm.at[s], srcv)
    pltpu.sync_copy(dst_hbm.at[s], dstv)
    pltpu.sync_copy(invd_hbm.at[s], invdv)

    def _stage(t, _):
        v = invdv[pl.ds(t * 16, 16)]
        for u in range(16):
            invds[t * 16 + u] = v[u]
        return 0
    lax.fori_loop(0, RPAD // 16, _stage, 0)

    def _zrow(r, _):
        for d4 in range(DH // 16):
            wbuf[r, pl.ds(d4 * 16, 16)] = jnp.zeros((16,), jnp.float32)
        return 0
    lax.fori_loop(0, C, _zrow, 0)

    def run_hop(gsrc, k):
        for q in range(NZ):
            pltpu.sync_copy(wbuf, acc.at[pl.ds(s * RT + q * C, C)])
        plsc.subcore_barrier()

        def gstart(j, u):
            pltpu.async_copy(gsrc.at[srcv.at[j]], bufs[u], gsems[u])

        def gwait(j, u):
            pltpu.make_async_copy(gsrc.at[srcv.at[j]], bufs[u],
                                  gsems[u]).wait()

        def sstart(j, u):
            pltpu.async_copy(bufs[u], acc.at[dstv.at[j]], ssems[u],
                             add=True)

        def swait(j, u):
            pltpu.make_async_copy(bufs[u], acc.at[dstv.at[j]],
                                  ssems[u]).wait()

        gstart(0, 0)
        gstart(1, 1)

        def _group(i, _):
            for u in range(4):
                j = 4 * i + u
                gwait(j, u)
                sstart(j, u)
                if u >= 2:
                    swait(j - 2, (u - 2) % 4)
                else:
                    @pl.when(j >= 2)
                    def _():
                        swait(j - 2, (u - 2) % 4)
                if u < 2:
                    gstart(j + 2, (u + 2) % 4)
                else:
                    @pl.when(j + 2 < NCH)
                    def _():
                        gstart(j + 2, (u + 2) % 4)
            return 0

        lax.fori_loop(0, NCH // 4, _group, 0)
        swait(NCH - 2, (NCH - 2) % 4)
        swait(NCH - 1, (NCH - 1) % 4)
        plsc.subcore_barrier()

        for q in range(NZ):
            base = s * RT + q * C
            pltpu.sync_copy(acc.at[pl.ds(base, C)], b0)

            def _srow(r, _):
                iv = invds[q * C + r]
                for d4 in range(DH // 16):
                    sl = pl.ds(d4 * 16, 16)
                    b0[r, sl] = b0[r, sl] * iv
                return 0
            lax.fori_loop(0, C, _srow, 0)
            pltpu.sync_copy(b0, gall_hbm.at[k, c, pl.ds(base, C)])
        plsc.subcore_barrier()

    run_hop(g0_hbm.at[c], 0)

    def _hop(k, _):
        run_hop(gall_hbm.at[k - 1, c], k)
        return 0
    lax.fori_loop(1, K, _hop, 0)


_NB = 2000
_GRID = N // _NB


def _tc_init_body(feat_ref, degp_ref, s_ref,
                  g_ref, out_ref, invd_ref, sqd_ref):
    degs = jnp.maximum(degp_ref[0, :, 0:1] + degp_ref[1, :, 0:1], 1.0)
    invd_ref[...] = 1.0 / degs
    sqd_ref[...] = jnp.sqrt(degs)
    norm = lax.rsqrt(degs)
    h = feat_ref[...]
    g = h * norm
    g_ref[0] = g[:, :DH]
    g_ref[1] = g[:, DH:]
    t = jnp.dot(h, s_ref[...], preferred_element_type=jnp.float32)
    out_ref[...] = jax.nn.sigmoid(t) * h


_tc_init = pl.pallas_call(
    _tc_init_body,
    grid=(_GRID,),
    in_specs=[
        pl.BlockSpec((_NB, D), lambda i: (i, 0)),
        pl.BlockSpec((NC, _NB, 16), lambda i: (0, i, 0)),
        pl.BlockSpec((D, 1), lambda i: (0, 0)),
    ],
    out_specs=[
        pl.BlockSpec((NC, _NB, DH), lambda i: (0, i, 0)),
        pl.BlockSpec((_NB, D), lambda i: (i, 0)),
        pl.BlockSpec((_NB, 1), lambda i: (i, 0)),
        pl.BlockSpec((_NB, 1), lambda i: (i, 0)),
    ],
    out_shape=[
        jax.ShapeDtypeStruct((NC, N, DH), jnp.float32),
        jax.ShapeDtypeStruct((N, D), jnp.float32),
        jax.ShapeDtypeStruct((N, 1), jnp.float32),
        jax.ShapeDtypeStruct((N, 1), jnp.float32),
    ],
)


def _tc_final_body(gall_ref, sqd_ref, s_ref, out0_ref, out_ref):
    k = pl.program_id(1)

    @pl.when(k == 0)
    def _():
        out_ref[...] = out0_ref[...]

    g = jnp.concatenate([gall_ref[0, 0], gall_ref[0, 1]], axis=-1)
    h = g * sqd_ref[...]
    t = jnp.dot(h, s_ref[...], preferred_element_type=jnp.float32)
    out_ref[...] += jax.nn.sigmoid(t) * h


_tc_final = pl.pallas_call(
    _tc_final_body,
    grid=(_GRID, K),
    in_specs=[
        pl.BlockSpec((1, NC, _NB, DH), lambda i, k: (k, 0, i, 0)),
        pl.BlockSpec((_NB, 1), lambda i, k: (i, 0)),
        pl.BlockSpec((D, 1), lambda i, k: (0, 0)),
        pl.BlockSpec((_NB, D), lambda i, k: (i, 0)),
    ],
    out_specs=pl.BlockSpec((_NB, D), lambda i, k: (i, 0)),
    out_shape=jax.ShapeDtypeStruct((N, D), jnp.float32),
)


def kernel(features, edge_index, s):
    src = edge_index[0].reshape(NS, NCH, C)
    dst = edge_index[1].reshape(NS, NCH, C)
    dstd = edge_index[1].reshape(NW, NCHD, C)
    degparts = _deg_kernel(dstd)
    g0, out0, invd, sqd = _tc_init(features, degparts, s)
    invdp = jnp.pad(invd[:, 0].reshape(NS, RT), ((0, 0), (0, RPAD - RT)))
    gall = _multihop_kernel(g0, src, dst, invdp)
    out = _tc_final(gall, sqd, s, out0)
    return out
